# baseline (speedup 1.0000x reference)
import functools

import jax
import jax.numpy as jnp
import numpy as np
from jax import lax
from jax.experimental import pallas as pl
from jax.experimental.pallas import tpu as pltpu
from jax.experimental.pallas import tpu_sc as plsc

_TEM = 6


def _diff_body(x_ref, wr_ref, wx_ref, dh_ref,
               dv0, dv1, dv2, dc0, dc1, dc2, *, pw, chans):
    ci = pl.program_id(1)
    nc = pl.num_programs(1)
    h = x_ref.shape[2]
    w = x_ref.shape[3]
    rc = 32
    dv_parts = (dv0, dv1, dv2)
    dc_parts = (dc0, dc1, dc2)

    @pl.when(ci == 0)
    def _z():
        for s in dv_parts + dc_parts:
            s[...] = jnp.zeros_like(s)
        dh_ref[...] = jnp.zeros_like(dh_ref)

    for rb in range(h // rc):
        r0 = rb * rc
        nv = rc if r0 + rc < h else rc - 1
        sv = sh = sc = None
        for c in range(chans):
            xc = x_ref[0, c, pl.ds(r0, min(rc + 1, h - r0)), :]
            d = xc[:nv, :] - xc[1:nv + 1, :]
            sv = d * d if sv is None else sv + d * d
            xr = xc[:rc, :]
            d = xr[:, :-1] - xr[:, 1:]
            sh = d * d if sh is None else sh + d * d
            d = xr[:, :-pw] - xr[:, pw:]
            sc = d * d if sc is None else sc + d * d
        for p in range(3):
            dv_parts[p][pl.ds(r0, nv), :] += sv[:, 2 * pw * p:2 * pw * (p + 1)]
        dh_ref[0, pl.ds(r0, rc), :] += sh
        dc_parts[0][pl.ds(r0, rc), :] += sc[:, :2 * pw]
        dc_parts[1][pl.ds(r0, rc), :] += sc[:, 2 * pw:4 * pw]
        dc_parts[2][pl.ds(r0, rc), :pw] += sc[:, 4 * pw:]

    @pl.when(ci == nc - 1)
    def _emit():
        hh = h // 2
        for p in range(3):
            e = dv_parts[p][pl.Slice(0, hh, 2), :]
            o = dv_parts[p][pl.Slice(1, hh, 2), :]
            wr_ref[0, 2 * p] = jnp.concatenate([e[:, :pw], o[:, :pw]], axis=1)
            wr_ref[0, 2 * p + 1] = jnp.concatenate([e[:, pw:], o[:, pw:]], axis=1)
        for p in range(3):
            e = dc_parts[p][pl.Slice(0, hh, 2), :]
            o = dc_parts[p][pl.Slice(1, hh, 2), :]
            wx_ref[0, 2 * p] = jnp.concatenate([e[:, :pw], o[:, :pw]], axis=1)
            if 2 * p + 1 < _TEM - 1:
                wx_ref[0, 2 * p + 1] = jnp.concatenate([e[:, pw:], o[:, pw:]], axis=1)


def _weights(x, chans):
    b, d, h, w = x.shape
    pw = w // _TEM
    grid = (b, d // chans)
    return pl.pallas_call(
        functools.partial(_diff_body, pw=pw, chans=chans),
        grid=grid,
        in_specs=[pl.BlockSpec((1, chans, h, w), lambda i, c: (i, c, 0, 0))],
        out_specs=[
            pl.BlockSpec((1, _TEM, h // 2, 2 * pw), lambda i, c: (i, 0, 0, 0)),
            pl.BlockSpec((1, _TEM - 1, h // 2, 2 * pw), lambda i, c: (i, 0, 0, 0)),
            pl.BlockSpec((1, h, w - 1), lambda i, c: (i, 0, 0)),
        ],
        out_shape=[
            jax.ShapeDtypeStruct((b, _TEM, h // 2, 2 * pw), jnp.float32),
            jax.ShapeDtypeStruct((b, _TEM - 1, h // 2, 2 * pw), jnp.float32),
            jax.ShapeDtypeStruct((b, h, w - 1), jnp.float32),
        ],
        scratch_shapes=[
            pltpu.VMEM((h, 2 * pw), jnp.float32),
            pltpu.VMEM((h, 2 * pw), jnp.float32),
            pltpu.VMEM((h, 2 * pw), jnp.float32),
            pltpu.VMEM((h, 2 * pw), jnp.float32),
            pltpu.VMEM((h, 2 * pw), jnp.float32),
            pltpu.VMEM((h, 2 * pw), jnp.float32),
        ],
        compiler_params=pltpu.CompilerParams(
            dimension_semantics=("parallel", "arbitrary"),
        ),
    )(x)


def _sc_index_body(out_hbm, scratch, sem, *, batch, height, width):
    pw = width // _TEM
    nw = 32
    wid = lax.axis_index("s") * 2 + lax.axis_index("c")
    e_row = (height - 1) * pw
    e_col = height * (pw - 1)
    lanes = lax.iota(jnp.int32, 16)

    def _rows(nr, r0, col0, L, cvec):
        def _row(i, _):
            a0 = (r0 + i) * width + col0
            for j in range((2 * L + 15) // 16):
                v = cvec + (a0 + 8 * j)
                for b in range(batch):
                    scratch[b, pl.ds(i * 2 * L + 16 * j, 16)] = v
            return 0
        lax.fori_loop(0, nr, _row, 0, unroll=False)

    def _copy(off, ln):
        pltpu.async_copy(scratch.at[:, pl.ds(0, ln)],
                         out_hbm.at[:, pl.ds(off, ln)], sem).wait()

    cvec = (lanes >> 1) + (lanes & 1) * width
    rpw = (height // nw)
    nlast = (height - 1) - rpw * (nw - 1)

    def _rowseg(t, _):
        r0 = wid * rpw

        @pl.when(wid < nw - 1)
        def _():
            _rows(rpw, r0, t * pw, pw, cvec)
            _copy(2 * e_row * t + 2 * pw * r0, 2 * pw * rpw)

        @pl.when(wid == nw - 1)
        def _():
            _rows(nlast, r0, t * pw, pw, cvec)
            _copy(2 * e_row * t + 2 * pw * r0, 2 * pw * nlast)
        return 0

    lax.fori_loop(0, _TEM, _rowseg, 0, unroll=False)

    cbase = 2 * _TEM * e_row
    blk = 64
    nunit = _TEM * (height // blk)

    def _colunit(u):
        t = u // (height // blk)
        p = u % (height // blk)
        r0 = p * blk
        col0 = t * pw
        nchunk = 2 * (pw - 1) * blk // 128

        tl = 2 * (pw - 1)

        bump = jnp.int32(width - (pw - 1))

        def _chunk(g, carry):
            r, mr = carry
            for j in range(8):
                c2t = mr + 16 * j + lanes
                v = ((r0 + r) * width + col0 + (c2t >> 1) + (c2t & 1)
                     + jnp.where(c2t >= tl, bump, jnp.int32(0)))
                for b in range(batch):
                    scratch[b, pl.ds(g * 128 + 16 * j, 16)] = v
            mr = mr + 128
            r = jnp.where(mr >= tl, r + 1, r)
            mr = jnp.where(mr >= tl, mr - tl, mr)
            r = jnp.where(mr >= tl, r + 1, r)
            mr = jnp.where(mr >= tl, mr - tl, mr)
            return (r, mr)

        lax.fori_loop(0, nchunk, _chunk, (0, 0), unroll=False)
        _copy(cbase + 2 * e_col * t + 2 * (pw - 1) * r0, 2 * (pw - 1) * blk)

    _colunit(wid)

    @pl.when(wid < nunit - nw)
    def _():
        _colunit(wid + nw)

    cvecx = (lanes >> 1) + (lanes & 1) * pw
    xbase = 2 * _TEM * (e_row + e_col)

    def _xseg(t, _):
        r0 = wid * rpw
        _rows(rpw, r0, t * pw, pw, cvecx)
        _copy(xbase + 2 * pw * height * t + 2 * pw * r0, 2 * pw * rpw)
        return 0

    lax.fori_loop(0, _TEM - 1, _xseg, 0, unroll=False)


def _sc_index(batch, height, width):
    pw = width // _TEM
    e_total = _TEM * ((height - 1) * pw + height * (pw - 1)) \
        + (_TEM - 1) * height * pw
    mesh = plsc.VectorSubcoreMesh(core_axis_name="c", subcore_axis_name="s")
    scr = 2 * (pw - 1) * 64
    k = functools.partial(
        pl.kernel, mesh=mesh,
        out_type=jax.ShapeDtypeStruct((batch, 2 * e_total), jnp.int32),
        scratch_types=[
            pltpu.VMEM((batch, scr), jnp.int32),
            pltpu.SemaphoreType.DMA,
        ],
    )(functools.partial(_sc_index_body, batch=batch, height=height,
                        width=width))
    return jnp.maximum(k().reshape(batch, e_total, 2), jnp.int32(0))


def _edge_index_host(height, width):
    row = np.arange(width, dtype=np.int32)[None, :]
    col = np.arange(height, dtype=np.int32)[:, None]
    raw = row + col * width
    pw = width // _TEM
    phases = [raw[:, i * pw:(i + 1) * pw] for i in range(_TEM)]
    rows, cols, cross = [], [], []
    for p in phases:
        rows.append(np.stack([p[:-1, :], p[1:, :]], axis=2).reshape(1, -1, 2))
        cols.append(np.stack([p[:, :-1], p[:, 1:]], axis=2).reshape(1, -1, 2))
    for i in range(_TEM - 1):
        cross.append(np.stack([phases[i], phases[i + 1]], axis=2).reshape(1, -1, 2))
    return np.concatenate(rows + cols + cross, axis=1)


def _edge_index(batch, height, width):
    idx = jnp.asarray(_edge_index_host(height, width))
    return jnp.broadcast_to(idx, (batch, idx.shape[1], 2))


def kernel(guide_in):
    b, d, h, w = guide_in.shape
    pw = w // _TEM
    wr, wx, dhm = _weights(guide_in, chans=16)
    nrow = (h - 1) * pw
    wr = wr.reshape(b, _TEM, (h // 2) * 2 * pw)[:, :, :nrow]
    wx = wx.reshape(b, _TEM - 1, h * pw)
    segs = []
    for t in range(_TEM):
        segs.append(wr[:, t])
        segs.append(dhm[:, :, t * pw:t * pw + pw - 1].reshape(b, -1))
    for t in range(_TEM - 1):
        segs.append(wx[:, t])
    weight = jnp.concatenate(segs, axis=1)
    index = _sc_index(b, h, w)
    return (index, weight)


# speedup vs baseline: 6.2572x; 6.2572x over previous
import functools

import jax
import jax.numpy as jnp
import numpy as np
from jax.experimental import pallas as pl
from jax.experimental.pallas import tpu as pltpu

_TEM = 6


def _diff_body(x_ref, wr_ref, wx_ref, dh_ref,
               dv0, dv1, dv2, dc0, dc1, dc2, *, pw, chans):
    ci = pl.program_id(1)
    nc = pl.num_programs(1)
    h = x_ref.shape[2]
    w = x_ref.shape[3]
    rc = 32
    dv_parts = (dv0, dv1, dv2)
    dc_parts = (dc0, dc1, dc2)

    @pl.when(ci == 0)
    def _z():
        for s in dv_parts + dc_parts:
            s[...] = jnp.zeros_like(s)
        dh_ref[...] = jnp.zeros_like(dh_ref)

    for rb in range(h // rc):
        r0 = rb * rc
        nv = rc if r0 + rc < h else rc - 1
        sv = sh = sc = None
        for c in range(chans):
            xc = x_ref[0, c, pl.ds(r0, min(rc + 1, h - r0)), :]
            d = xc[:nv, :] - xc[1:nv + 1, :]
            sv = d * d if sv is None else sv + d * d
            xr = xc[:rc, :]
            d = xr[:, :-1] - xr[:, 1:]
            sh = d * d if sh is None else sh + d * d
            d = xr[:, :-pw] - xr[:, pw:]
            sc = d * d if sc is None else sc + d * d
        for p in range(3):
            dv_parts[p][pl.ds(r0, nv), :] += sv[:, 2 * pw * p:2 * pw * (p + 1)]
        dh_ref[0, pl.ds(r0, rc), :] += sh
        dc_parts[0][pl.ds(r0, rc), :] += sc[:, :2 * pw]
        dc_parts[1][pl.ds(r0, rc), :] += sc[:, 2 * pw:4 * pw]
        dc_parts[2][pl.ds(r0, rc), :pw] += sc[:, 4 * pw:]

    @pl.when(ci == nc - 1)
    def _emit():
        hh = h // 2
        for p in range(3):
            e = dv_parts[p][pl.Slice(0, hh, 2), :]
            o = dv_parts[p][pl.Slice(1, hh, 2), :]
            wr_ref[0, 2 * p] = jnp.concatenate([e[:, :pw], o[:, :pw]], axis=1)
            wr_ref[0, 2 * p + 1] = jnp.concatenate([e[:, pw:], o[:, pw:]], axis=1)
        for p in range(3):
            e = dc_parts[p][pl.Slice(0, hh, 2), :]
            o = dc_parts[p][pl.Slice(1, hh, 2), :]
            wx_ref[0, 2 * p] = jnp.concatenate([e[:, :pw], o[:, :pw]], axis=1)
            if 2 * p + 1 < _TEM - 1:
                wx_ref[0, 2 * p + 1] = jnp.concatenate([e[:, pw:], o[:, pw:]], axis=1)


def _weights(x, chans):
    b, d, h, w = x.shape
    pw = w // _TEM
    grid = (b, d // chans)
    return pl.pallas_call(
        functools.partial(_diff_body, pw=pw, chans=chans),
        grid=grid,
        in_specs=[pl.BlockSpec((1, chans, h, w), lambda i, c: (i, c, 0, 0))],
        out_specs=[
            pl.BlockSpec((1, _TEM, h // 2, 2 * pw), lambda i, c: (i, 0, 0, 0)),
            pl.BlockSpec((1, _TEM - 1, h // 2, 2 * pw), lambda i, c: (i, 0, 0, 0)),
            pl.BlockSpec((1, h, w - 1), lambda i, c: (i, 0, 0)),
        ],
        out_shape=[
            jax.ShapeDtypeStruct((b, _TEM, h // 2, 2 * pw), jnp.float32),
            jax.ShapeDtypeStruct((b, _TEM - 1, h // 2, 2 * pw), jnp.float32),
            jax.ShapeDtypeStruct((b, h, w - 1), jnp.float32),
        ],
        scratch_shapes=[
            pltpu.VMEM((h, 2 * pw), jnp.float32),
            pltpu.VMEM((h, 2 * pw), jnp.float32),
            pltpu.VMEM((h, 2 * pw), jnp.float32),
            pltpu.VMEM((h, 2 * pw), jnp.float32),
            pltpu.VMEM((h, 2 * pw), jnp.float32),
            pltpu.VMEM((h, 2 * pw), jnp.float32),
        ],
        compiler_params=pltpu.CompilerParams(
            dimension_semantics=("parallel", "arbitrary"),
        ),
    )(x)


def _edge_index_host(height, width):
    row = np.arange(width, dtype=np.int32)[None, :]
    col = np.arange(height, dtype=np.int32)[:, None]
    raw = row + col * width
    pw = width // _TEM
    phases = [raw[:, i * pw:(i + 1) * pw] for i in range(_TEM)]
    rows, cols, cross = [], [], []
    for p in phases:
        rows.append(np.stack([p[:-1, :], p[1:, :]], axis=2).reshape(1, -1, 2))
        cols.append(np.stack([p[:, :-1], p[:, 1:]], axis=2).reshape(1, -1, 2))
    for i in range(_TEM - 1):
        cross.append(np.stack([phases[i], phases[i + 1]], axis=2).reshape(1, -1, 2))
    return np.concatenate(rows + cols + cross, axis=1)


def _edge_index(batch, height, width):
    idx = jnp.asarray(_edge_index_host(height, width))
    return jnp.broadcast_to(idx, (batch, idx.shape[1], 2))


def kernel(guide_in):
    b, d, h, w = guide_in.shape
    pw = w // _TEM
    wr, wx, dhm = _weights(guide_in, chans=16)
    nrow = (h - 1) * pw
    wr = wr.reshape(b, _TEM, (h // 2) * 2 * pw)[:, :, :nrow]
    wx = wx.reshape(b, _TEM - 1, h * pw)
    segs = []
    for t in range(_TEM):
        segs.append(wr[:, t])
        segs.append(dhm[:, :, t * pw:t * pw + pw - 1].reshape(b, -1))
    for t in range(_TEM - 1):
        segs.append(wx[:, t])
    weight = jnp.concatenate(segs, axis=1)
    index = _edge_index(b, h, w)
    return (index, weight)
